# baseline (speedup 1.0000x reference)
import jax
import jax.numpy as jnp
from jax import lax
from jax.experimental import pallas as pl
from jax.experimental.pallas import tpu as pltpu
from jax.experimental.pallas import tpu_sc as plsc

_HID = 32
_NW = 32
_CHUNK = 256
_SUB = 128
_LANE = 128
_STAGE = 64


def _sc_body(gidx_hbm, meta_hbm, int_hbm, flt_hbm, out_hbm,
             gidx_v, meta_v, rows_v, comp_v, flt_v,
             gsems, ssems, fsem):
    n_rows_in = gidx_hbm.shape[0]
    n_per_w = n_rows_in * _LANE // _NW
    rows_per_w = n_per_w // _LANE
    n_chunks = n_per_w // _CHUNK
    wid = lax.axis_index("s") * 2 + lax.axis_index("c")
    base_w = pl.multiple_of(wid * n_per_w, n_per_w)

    flt_cp = pltpu.async_copy(flt_hbm, flt_v, fsem)

    row0 = wid * rows_per_w
    base_al = jnp.minimum((row0 >> 3) << 3, n_rows_in - _STAGE)
    base_al = pl.multiple_of(base_al, 8)
    off = row0 - base_al
    pltpu.sync_copy(gidx_hbm.at[pl.ds(base_al, _STAGE)], gidx_v)
    pltpu.sync_copy(meta_hbm.at[pl.ds(base_al, _STAGE)], meta_v)

    def fire_gather(ci, slot):
        for k in range(_CHUNK // _SUB):
            idx_row = off + ci * (_CHUNK // _LANE) + k
            rsl = pl.ds(k * _SUB, _SUB)
            pltpu.async_copy(
                int_hbm.at[gidx_v.at[idx_row]], rows_v.at[slot].at[rsl],
                gsems.at[slot])

    def drain_gather(slot):
        pltpu.make_async_copy(
            int_hbm.at[pl.ds(0, _CHUNK)], rows_v.at[slot],
            gsems.at[slot]).wait()

    def drain_store(slot):
        pltpu.make_async_copy(
            out_hbm.at[pl.ds(0, _CHUNK // 4)], comp_v.at[slot],
            ssems.at[slot]).wait()

    def compact(ci, slot):
        def grp_body(g, carry):
            mrow = off + ci * (_CHUNK // _LANE) + lax.shift_right_logical(g, 3)
            mcol = lax.shift_left(g & 7, 4)
            m16 = meta_v[mrow, pl.ds(mcol, 16)]
            cb16 = m16 & 0xFF
            fr16 = lax.shift_right_logical(m16, 16)
            for j in range(16):
                t = g * 16 + j
                cb = cb16[j]
                fr = fr16[j]
                for k in range(2):
                    acc = (rows_v[slot, t, pl.ds(cb + k * 16, 16)]
                           + flt_v[fr, pl.ds(k * 16, 16)])
                    comp_v[slot, g * 4 + j // 4,
                           pl.ds((j % 4) * 32 + k * 16, 16)] = acc
            return carry

        lax.fori_loop(0, _CHUNK // 16, grp_body, 0)

    def fire_store(ci, slot):
        rows_out = _CHUNK // 4
        base = pl.multiple_of(base_w // 4 + ci * rows_out, rows_out)
        pltpu.async_copy(
            comp_v.at[slot], out_hbm.at[pl.ds(base, rows_out)],
            ssems.at[slot])

    flt_cp.wait()
    fire_gather(0, 0)

    def loop_body(ci2, carry):
        a = ci2 * 2
        b = a + 1
        fire_gather(b, 1)
        drain_gather(0)

        @pl.when(ci2 > 0)
        def _():
            drain_store(0)

        compact(a, 0)
        fire_store(a, 0)
        fire_gather(a + 2, 0)
        drain_gather(1)

        @pl.when(ci2 > 0)
        def _():
            drain_store(1)

        compact(b, 1)
        fire_store(b, 1)
        return carry

    lax.fori_loop(0, (n_chunks - 1) // 2, loop_body, 0)

    tail = n_chunks - 1
    drain_gather(0)
    drain_store(0)
    compact(tail, 0)
    fire_store(tail, 0)
    drain_store(1)
    drain_store(0)


def kernel(input, int_table, float_table):
    b, l = input.shape
    n = b * l
    ii = input.astype(jnp.int32)
    fr = ((input - ii.astype(jnp.float32)) * 100.0).astype(jnp.int32)
    zi = lax.optimization_barrier(jnp.zeros((), jnp.int32))
    gidx = lax.shift_right_logical(ii, 2).reshape(n // _LANE, _LANE) | zi
    meta = (lax.shift_left(ii & 3, 5)
            | lax.shift_left(fr, 16)).reshape(n // _LANE, _LANE) | zi
    int_wide = int_table.reshape(int_table.shape[0] // 4, 4 * _HID)
    flt_pad = jnp.zeros((104, _LANE), jnp.float32).at[:100, :_HID].set(
        float_table)
    mesh = plsc.VectorSubcoreMesh(core_axis_name="c", subcore_axis_name="s")
    run = pl.kernel(
        _sc_body,
        out_type=jax.ShapeDtypeStruct((n // 4, 4 * _HID), jnp.float32),
        mesh=mesh,
        scratch_types=[
            pltpu.VMEM((_STAGE, _LANE), jnp.int32),
            pltpu.VMEM((_STAGE, _LANE), jnp.int32),
            pltpu.VMEM((2, _CHUNK, 4 * _HID), jnp.float32),
            pltpu.VMEM((2, _CHUNK // 4, 4 * _HID), jnp.float32),
            pltpu.VMEM((104, _LANE), jnp.float32),
            pltpu.SemaphoreType.DMA((2,)),
            pltpu.SemaphoreType.DMA((2,)),
            pltpu.SemaphoreType.DMA,
        ],
    )
    out = run(gidx, meta, int_wide, flt_pad)
    zf = lax.optimization_barrier(jnp.zeros((), jnp.float32))
    return out.reshape(b, l, _HID) + zf


# speedup vs baseline: 1.0271x; 1.0271x over previous
import jax
import jax.numpy as jnp
from jax import lax
from jax.experimental import pallas as pl
from jax.experimental.pallas import tpu as pltpu
from jax.experimental.pallas import tpu_sc as plsc

_HID = 32
_NW = 32
_CHUNK = 256
_SUB = 128
_LANE = 128
_STAGE = 64


def _sc_body(gidx_hbm, meta_hbm, int_hbm, flt_hbm, out_hbm,
             gidx_v, meta_v, rows_v, comp_v, flt_v,
             gsems, ssems, fsem):
    n_rows_in = gidx_hbm.shape[0]
    n_per_w = n_rows_in * _LANE // _NW
    rows_per_w = n_per_w // _LANE
    n_chunks = n_per_w // _CHUNK
    wid = lax.axis_index("s") * 2 + lax.axis_index("c")
    base_w = pl.multiple_of(wid * n_per_w, n_per_w)

    flt_cp = pltpu.async_copy(flt_hbm, flt_v, fsem)

    row0 = wid * rows_per_w
    base_al = jnp.minimum((row0 >> 3) << 3, n_rows_in - _STAGE)
    base_al = pl.multiple_of(base_al, 8)
    off = row0 - base_al
    pltpu.sync_copy(gidx_hbm.at[pl.ds(base_al, _STAGE)], gidx_v)
    pltpu.sync_copy(meta_hbm.at[pl.ds(base_al, _STAGE)], meta_v)

    def fire_gather(ci, slot):
        for k in range(_CHUNK // _SUB):
            idx_row = off + ci * (_CHUNK // _LANE) + k
            rsl = pl.ds(k * _SUB, _SUB)
            pltpu.async_copy(
                int_hbm.at[gidx_v.at[idx_row]], rows_v.at[slot].at[rsl],
                gsems.at[slot])

    def drain_gather(slot):
        pltpu.make_async_copy(
            int_hbm.at[pl.ds(0, _CHUNK)], rows_v.at[slot],
            gsems.at[slot]).wait()

    def drain_store(slot):
        pltpu.make_async_copy(
            out_hbm.at[pl.ds(0, _CHUNK // 4)], comp_v.at[slot],
            ssems.at[slot]).wait()

    def compact(ci, slot):
        def grp_body(g, carry):
            mrow = off + ci * (_CHUNK // _LANE) + lax.shift_right_logical(g, 3)
            mcol = lax.shift_left(g & 7, 4)
            m16 = meta_v[mrow, pl.ds(mcol, 16)]
            cb16 = m16 & 0xFF
            fr16 = lax.shift_right_logical(m16, 16)
            for j in range(16):
                t = g * 16 + j
                cb = cb16[j]
                fr = fr16[j]
                for k in range(2):
                    acc = (rows_v[slot, t, pl.ds(cb + k * 16, 16)]
                           + flt_v[fr, pl.ds(k * 16, 16)])
                    comp_v[slot, g * 4 + j // 4,
                           pl.ds((j % 4) * 32 + k * 16, 16)] = acc
            return carry

        lax.fori_loop(0, _CHUNK // 16, grp_body, 0)

    def fire_store(ci, slot):
        rows_out = _CHUNK // 4
        base = pl.multiple_of(base_w // 4 + ci * rows_out, rows_out)
        pltpu.async_copy(
            comp_v.at[slot], out_hbm.at[pl.ds(base, rows_out)],
            ssems.at[slot])

    flt_cp.wait()
    fire_gather(0, 0)

    def loop_body(ci2, carry):
        a = ci2 * 2
        b = a + 1
        fire_gather(b, 1)
        drain_gather(0)

        @pl.when(ci2 > 0)
        def _():
            drain_store(0)

        compact(a, 0)
        fire_store(a, 0)
        fire_gather(a + 2, 0)
        drain_gather(1)

        @pl.when(ci2 > 0)
        def _():
            drain_store(1)

        compact(b, 1)
        fire_store(b, 1)
        return carry

    lax.fori_loop(0, (n_chunks - 1) // 2, loop_body, 0)

    tail = n_chunks - 1
    drain_gather(0)
    drain_store(0)
    compact(tail, 0)
    fire_store(tail, 0)
    drain_store(1)
    drain_store(0)


def kernel(input, int_table, float_table):
    b, l = input.shape
    n = b * l
    ii = input.astype(jnp.int32)
    fr = ((input - ii.astype(jnp.float32)) * 100.0).astype(jnp.int32)
    gidx = lax.shift_right_logical(ii, 2).reshape(n // _LANE, _LANE)
    meta = (lax.shift_left(ii & 3, 5)
            | lax.shift_left(fr, 16)).reshape(n // _LANE, _LANE)
    int_wide = int_table.reshape(int_table.shape[0] // 4, 4 * _HID)
    flt_pad = jnp.zeros((104, _LANE), jnp.float32).at[:100, :_HID].set(
        float_table)
    mesh = plsc.VectorSubcoreMesh(core_axis_name="c", subcore_axis_name="s")
    run = pl.kernel(
        _sc_body,
        out_type=jax.ShapeDtypeStruct((n // 4, 4 * _HID), jnp.float32),
        mesh=mesh,
        scratch_types=[
            pltpu.VMEM((_STAGE, _LANE), jnp.int32),
            pltpu.VMEM((_STAGE, _LANE), jnp.int32),
            pltpu.VMEM((2, _CHUNK, 4 * _HID), jnp.float32),
            pltpu.VMEM((2, _CHUNK // 4, 4 * _HID), jnp.float32),
            pltpu.VMEM((104, _LANE), jnp.float32),
            pltpu.SemaphoreType.DMA((2,)),
            pltpu.SemaphoreType.DMA((2,)),
            pltpu.SemaphoreType.DMA,
        ],
    )
    out = run(gidx, meta, int_wide, flt_pad)
    return out.reshape(b, l, _HID)
